# baseline (speedup 1.0000x reference)
import jax
import jax.numpy as jnp
from jax.experimental import pallas as pl

_B, _N, _DIM = 32, 8192, 128
_K = 32
_ROWS = _N // 128
_BC = 4
_HCH = 4096


def _fast_exp(t):
    u = jnp.maximum(t, -30.0) * (1.0 / 128.0)
    p = 1.0 + u * (1.0 + u * (0.5 + u * (1.0 / 6.0 + u * (
        1.0 / 24.0 + u * (1.0 / 120.0 + u * (1.0 / 720.0))))))
    for _ in range(7):
        p = p * p
    return p


def _wta_kernel(x_ref, w_ref, b_ref, out_ref, mask_ref):
    nh = _N // _HCH
    bbar = jnp.mean(b_ref[...])
    s_list = []
    for cb in range(_BC):
        s_parts = []
        for h in range(nh):
            proj = jax.lax.dot_general(
                x_ref[cb, h * _HCH:(h + 1) * _HCH, :], w_ref[...],
                dimension_numbers=(((1,), (1,)), ((), ())),
                preferred_element_type=jnp.float32,
            )
            proj3 = proj.reshape(_HCH // 128, 128, _DIM)
            s_parts.append(jnp.mean(proj3, axis=-1) + bbar)
        s_list.append(jnp.concatenate(s_parts, axis=0))
    s4 = jnp.stack(s_list)

    m0 = jnp.max(s4, axis=(1, 2), keepdims=True)
    e4 = _fast_exp(s4 - m0)
    z4 = jnp.sum(e4, axis=(1, 2), keepdims=True)

    big = jnp.int32(2 ** 30)
    neg = jnp.float32(-jnp.inf)
    noct = _ROWS // 8
    s8 = s4.reshape(_BC, noct, 8, 128)
    lin8 = (jax.lax.broadcasted_iota(jnp.int32, (1, noct, 8, 128), 1) * 1024
            + jax.lax.broadcasted_iota(jnp.int32, (1, noct, 8, 128), 2) * 128
            + jax.lax.broadcasted_iota(jnp.int32, (1, noct, 8, 128), 3))
    kslot = jax.lax.broadcasted_iota(jnp.int32, (1, 1, 128), 2)

    def oct_body(k, carry):
        sw, cval, clin = carry
        m = jnp.max(sw, axis=(2, 3), keepdims=True)
        idx = jnp.min(jnp.where(sw == m, lin8, big),
                      axis=(2, 3), keepdims=True)
        sw = jnp.where(lin8 == idx, neg, sw)
        sel = kslot == k
        cval = jnp.where(sel, m[:, :, :, 0], cval)
        clin = jnp.where(sel, idx[:, :, :, 0], clin)
        return sw, cval, clin

    _, cval, clin = jax.lax.fori_loop(
        0, _K, oct_body,
        (s8,
         jnp.full((_BC, noct, 128), neg),
         jnp.full((_BC, noct, 128), big)))

    def merge_body(_, carry):
        cv, vstar, istar = carry
        m = jnp.max(cv, axis=(1, 2), keepdims=True)
        idx = jnp.min(jnp.where(cv == m, clin, big),
                      axis=(1, 2), keepdims=True)
        cv = jnp.where(clin == idx, neg, cv)
        return cv, m, idx

    _, vstar, istar = jax.lax.fori_loop(
        0, _K, merge_body,
        (cval, jnp.zeros((_BC, 1, 1), jnp.float32),
         jnp.zeros((_BC, 1, 1), jnp.int32)))

    lin = (jax.lax.broadcasted_iota(jnp.int32, (1, _ROWS, 128), 1) * 128
           + jax.lax.broadcasted_iota(jnp.int32, (1, _ROWS, 128), 2))
    keep = (s4 > vstar) | ((s4 == vstar) & (lin <= istar))
    msk4 = jnp.where(keep, 1.0, 0.0)

    w4 = e4 * msk4
    rh = _HCH // 128
    for cb in range(_BC):
        acc = jnp.zeros((_DIM,), jnp.float32)
        for h in range(nh):
            x3 = x_ref[cb, h * _HCH:(h + 1) * _HCH, :].reshape(rh, 128, _DIM)
            wh = w4[cb, h * rh:(h + 1) * rh]
            prod = (x3 * wh[:, :, None]).reshape(_HCH, _DIM)
            acc = acc + jnp.sum(prod, axis=0)
        out_ref[cb, 0, :] = acc * (1.0 / z4[cb, 0, 0])
    mask_ref[...] = msk4


def kernel(x, W, b):
    out, mask3 = pl.pallas_call(
        _wta_kernel,
        grid=(_B // _BC,),
        in_specs=[
            pl.BlockSpec((_BC, _N, _DIM), lambda i: (i, 0, 0)),
            pl.BlockSpec((_DIM, _DIM), lambda i: (0, 0)),
            pl.BlockSpec((_DIM,), lambda i: (0,)),
        ],
        out_specs=[
            pl.BlockSpec((_BC, 1, _DIM), lambda i: (i, 0, 0)),
            pl.BlockSpec((_BC, _ROWS, 128), lambda i: (i, 0, 0)),
        ],
        out_shape=[
            jax.ShapeDtypeStruct((_B, 1, _DIM), jnp.float32),
            jax.ShapeDtypeStruct((_B, _ROWS, 128), jnp.float32),
        ],
    )(x, W, b)
    return out.reshape(_B, _DIM), mask3.reshape(_B, _N)


# speedup vs baseline: 1.4548x; 1.4548x over previous
import jax
import jax.numpy as jnp
from jax.experimental import pallas as pl

_B, _N, _DIM = 32, 8192, 128
_K = 32
_ROWS = _N // 128
_BC = 4
_HCH = 4096


def _fast_exp(t):
    u = jnp.maximum(t, -30.0) * (1.0 / 128.0)
    p = 1.0 + u * (1.0 + u * (0.5 + u * (1.0 / 6.0 + u * (
        1.0 / 24.0 + u * (1.0 / 120.0 + u * (1.0 / 720.0))))))
    for _ in range(7):
        p = p * p
    return p


def _wta_kernel(x_ref, w_ref, b_ref, out_ref, mask_ref):
    nh = _N // _HCH
    bbar = jnp.mean(b_ref[...])
    s_list = []
    for cb in range(_BC):
        s_parts = []
        for h in range(nh):
            proj = jax.lax.dot_general(
                x_ref[cb, h * _HCH:(h + 1) * _HCH, :], w_ref[...],
                dimension_numbers=(((1,), (1,)), ((), ())),
                preferred_element_type=jnp.float32,
            )
            proj3 = proj.reshape(_HCH // 128, 128, _DIM)
            s_parts.append(jnp.mean(proj3, axis=-1) + bbar)
        s_list.append(jnp.concatenate(s_parts, axis=0))
    s4 = jnp.stack(s_list)

    m0 = jnp.max(s4, axis=(1, 2), keepdims=True)
    e4 = _fast_exp(s4 - m0)
    z4 = jnp.sum(e4, axis=(1, 2), keepdims=True)

    big = jnp.int32(2 ** 30)
    neg = jnp.float32(-jnp.inf)
    lin = (jax.lax.broadcasted_iota(jnp.int32, (1, _ROWS, 128), 1) * 128
           + jax.lax.broadcasted_iota(jnp.int32, (1, _ROWS, 128), 2))

    def body(_, carry):
        sw, vstar, istar = carry
        m = jnp.max(sw, axis=(1, 2), keepdims=True)
        idx = jnp.min(jnp.where(sw == m, lin, big),
                      axis=(1, 2), keepdims=True)
        sw = jnp.where(lin == idx, neg, sw)
        return sw, m, idx

    _, vstar, istar = jax.lax.fori_loop(
        0, _K, body,
        (s4, jnp.zeros((_BC, 1, 1), jnp.float32),
         jnp.zeros((_BC, 1, 1), jnp.int32)))

    keep = (s4 > vstar) | ((s4 == vstar) & (lin <= istar))
    msk4 = jnp.where(keep, 1.0, 0.0)

    w4 = e4 * msk4
    rh = _HCH // 128
    for cb in range(_BC):
        acc = jnp.zeros((_DIM,), jnp.float32)
        for h in range(nh):
            x3 = x_ref[cb, h * _HCH:(h + 1) * _HCH, :].reshape(rh, 128, _DIM)
            wh = w4[cb, h * rh:(h + 1) * rh]
            prod = (x3 * wh[:, :, None]).reshape(_HCH, _DIM)
            acc = acc + jnp.sum(prod, axis=0)
        out_ref[cb, 0, :] = acc * (1.0 / z4[cb, 0, 0])
    mask_ref[...] = msk4


def kernel(x, W, b):
    out, mask3 = pl.pallas_call(
        _wta_kernel,
        grid=(_B // _BC,),
        in_specs=[
            pl.BlockSpec((_BC, _N, _DIM), lambda i: (i, 0, 0)),
            pl.BlockSpec((_DIM, _DIM), lambda i: (0, 0)),
            pl.BlockSpec((_DIM,), lambda i: (0,)),
        ],
        out_specs=[
            pl.BlockSpec((_BC, 1, _DIM), lambda i: (i, 0, 0)),
            pl.BlockSpec((_BC, _ROWS, 128), lambda i: (i, 0, 0)),
        ],
        out_shape=[
            jax.ShapeDtypeStruct((_B, 1, _DIM), jnp.float32),
            jax.ShapeDtypeStruct((_B, _ROWS, 128), jnp.float32),
        ],
    )(x, W, b)
    return out.reshape(_B, _DIM), mask3.reshape(_B, _N)


# speedup vs baseline: 1.9034x; 1.3083x over previous
import jax
import jax.numpy as jnp
from jax.experimental import pallas as pl

_B, _N, _DIM = 32, 8192, 128
_K = 32
_ROWS = _N // 128
_BC = 4
_HCH = 4096


def _fast_exp(t):
    u = jnp.maximum(t, -30.0) * (1.0 / 128.0)
    p = 1.0 + u * (1.0 + u * (0.5 + u * (1.0 / 6.0 + u * (
        1.0 / 24.0 + u * (1.0 / 120.0 + u * (1.0 / 720.0))))))
    for _ in range(7):
        p = p * p
    return p


def _wta_kernel(x_ref, w_ref, b_ref, out_ref, mask_ref):
    nh = _N // _HCH
    bbar = jnp.mean(b_ref[...])
    s_list = []
    for cb in range(_BC):
        s_parts = []
        for h in range(nh):
            proj = jax.lax.dot_general(
                x_ref[cb, h * _HCH:(h + 1) * _HCH, :], w_ref[...],
                dimension_numbers=(((1,), (1,)), ((), ())),
                preferred_element_type=jnp.float32,
            )
            proj3 = proj.reshape(_HCH // 128, 128, _DIM)
            s_parts.append(jnp.mean(proj3, axis=-1) + bbar)
        s_list.append(jnp.concatenate(s_parts, axis=0))
    s4 = jnp.stack(s_list)

    m0 = jnp.max(s4, axis=(1, 2), keepdims=True)
    e4 = _fast_exp(s4 - m0)
    z4 = jnp.sum(e4, axis=(1, 2), keepdims=True)

    lin = (jax.lax.broadcasted_iota(jnp.int32, (1, _ROWS, 128), 1) * 128
           + jax.lax.broadcasted_iota(jnp.int32, (1, _ROWS, 128), 2))
    one = jnp.int32(1)
    zero = jnp.int32(0)

    si = jax.lax.bitcast_convert_type(s4 + 0.0, jnp.int32)
    si = jnp.where(si < 0, si ^ jnp.int32(0x7FFFFFFF), si)

    p = jnp.zeros((_BC, 1, 1), jnp.int32)
    for r, shift in enumerate((28, 24, 20, 16, 12, 8, 4, 0)):
        js = range(-8, 8) if r == 0 else range(16)
        nsat = jnp.zeros((_BC, 1, 1), jnp.int32)
        for j in js:
            t = p + jnp.int32(j * (1 << shift))
            cnt = jnp.sum(jnp.where(si >= t, one, zero),
                          axis=(1, 2), keepdims=True)
            nsat = nsat + jnp.where(cnt >= _K, one, zero)
        jstar = nsat - (9 if r == 0 else 1)
        p = p + jstar * jnp.int32(1 << shift)

    eq = si == p
    cnt_gt = jnp.sum(jnp.where(si > p, one, zero), axis=(1, 2), keepdims=True)
    jneed = _K - cnt_gt
    pi = jnp.zeros((_BC, 1, 1), jnp.int32)
    for shift in (12, 8, 4, 0):
        step = 1 << shift
        nsat = jnp.zeros((_BC, 1, 1), jnp.int32)
        for n in range(16):
            c = jnp.sum(jnp.where(eq & (lin < pi + jnp.int32((n + 1) * step)),
                                  one, zero), axis=(1, 2), keepdims=True)
            nsat = nsat + jnp.where(c >= jneed, one, zero)
        pi = pi + (jnp.int32(16) - nsat) * jnp.int32(step)
    istar = pi

    keep = (si > p) | (eq & (lin <= istar))
    msk4 = jnp.where(keep, 1.0, 0.0)

    w4 = e4 * msk4
    rh = _HCH // 128
    for cb in range(_BC):
        acc = jnp.zeros((_DIM,), jnp.float32)
        for h in range(nh):
            x3 = x_ref[cb, h * _HCH:(h + 1) * _HCH, :].reshape(rh, 128, _DIM)
            wh = w4[cb, h * rh:(h + 1) * rh]
            prod = (x3 * wh[:, :, None]).reshape(_HCH, _DIM)
            acc = acc + jnp.sum(prod, axis=0)
        out_ref[cb, 0, :] = acc * (1.0 / z4[cb, 0, 0])
    mask_ref[...] = msk4


def kernel(x, W, b):
    out, mask3 = pl.pallas_call(
        _wta_kernel,
        grid=(_B // _BC,),
        in_specs=[
            pl.BlockSpec((_BC, _N, _DIM), lambda i: (i, 0, 0)),
            pl.BlockSpec((_DIM, _DIM), lambda i: (0, 0)),
            pl.BlockSpec((_DIM,), lambda i: (0,)),
        ],
        out_specs=[
            pl.BlockSpec((_BC, 1, _DIM), lambda i: (i, 0, 0)),
            pl.BlockSpec((_BC, _ROWS, 128), lambda i: (i, 0, 0)),
        ],
        out_shape=[
            jax.ShapeDtypeStruct((_B, 1, _DIM), jnp.float32),
            jax.ShapeDtypeStruct((_B, _ROWS, 128), jnp.float32),
        ],
    )(x, W, b)
    return out.reshape(_B, _DIM), mask3.reshape(_B, _N)


# speedup vs baseline: 2.2776x; 1.1966x over previous
import jax
import jax.numpy as jnp
from jax.experimental import pallas as pl

_B, _N, _DIM = 32, 8192, 128
_K = 32
_BC = 4
_NH = 2
_HCH = _N // _NH
_NR = _BC * _NH


def _fast_exp(t):
    u = jnp.maximum(t, -30.0) * (1.0 / 128.0)
    p = 1.0 + u * (1.0 + u * (0.5 + u * (1.0 / 6.0 + u * (
        1.0 / 24.0 + u * (1.0 / 120.0 + u * (1.0 / 720.0))))))
    for _ in range(7):
        p = p * p
    return p


def _bsum(v):
    r = jnp.sum(v, axis=1, keepdims=True)
    r3 = r.reshape(_BC, _NH, 1)
    t = jnp.sum(r3, axis=1, keepdims=True)
    return jnp.broadcast_to(t, (_BC, _NH, 1)).reshape(_NR, 1)


def _wta_kernel(x_ref, w_ref, b_ref, out_ref, mask_ref):
    bbar = jnp.mean(b_ref[...])
    rows = []
    for cb in range(_BC):
        for h in range(_NH):
            projt = jax.lax.dot_general(
                w_ref[...], x_ref[cb, h * _HCH:(h + 1) * _HCH, :],
                dimension_numbers=(((1,), (1,)), ((), ())),
                preferred_element_type=jnp.float32,
            )
            rows.append(jnp.sum(projt, axis=0, keepdims=True) * (1.0 / _DIM)
                        + bbar)
    s = jnp.concatenate(rows, axis=0)

    rmax = jnp.max(s, axis=1, keepdims=True).reshape(_BC, _NH, 1)
    m0 = jnp.broadcast_to(jnp.max(rmax, axis=1, keepdims=True),
                          (_BC, _NH, 1)).reshape(_NR, 1)
    e = _fast_exp(s - m0)
    z = _bsum(e)

    lin = (jax.lax.broadcasted_iota(jnp.int32, (_NR, _HCH), 0) % _NH) * _HCH \
        + jax.lax.broadcasted_iota(jnp.int32, (_NR, _HCH), 1)
    one = jnp.int32(1)
    zero = jnp.int32(0)

    si = jax.lax.bitcast_convert_type(s + 0.0, jnp.int32)
    si = jnp.where(si < 0, si ^ jnp.int32(0x7FFFFFFF), si)

    p = jnp.zeros((_NR, 1), jnp.int32)
    for r, shift in enumerate((28, 24, 20, 16, 12, 8, 4, 0)):
        js = range(-8, 8) if r == 0 else range(16)
        nsat = jnp.zeros((_NR, 1), jnp.int32)
        for j in js:
            t = p + jnp.int32(j * (1 << shift))
            cnt = _bsum(jnp.where(si >= t, one, zero))
            nsat = nsat + jnp.where(cnt >= _K, one, zero)
        jstar = nsat - (9 if r == 0 else 1)
        p = p + jstar * jnp.int32(1 << shift)

    eq = si == p
    cnt_gt = _bsum(jnp.where(si > p, one, zero))
    jneed = _K - cnt_gt
    pi = jnp.zeros((_NR, 1), jnp.int32)
    for shift in (12, 8, 4, 0):
        step = 1 << shift
        nsat = jnp.zeros((_NR, 1), jnp.int32)
        for n in range(16):
            c = _bsum(jnp.where(eq & (lin < pi + jnp.int32((n + 1) * step)),
                                one, zero))
            nsat = nsat + jnp.where(c >= jneed, one, zero)
        pi = pi + (jnp.int32(16) - nsat) * jnp.int32(step)

    keep = (si > p) | (eq & (lin <= pi))
    msk = jnp.where(keep, 1.0, 0.0)
    w = e * msk

    for cb in range(_BC):
        acc = jnp.zeros((1, _DIM), jnp.float32)
        for h in range(_NH):
            r = cb * _NH + h
            acc = acc + jax.lax.dot_general(
                w[r:r + 1, :], x_ref[cb, h * _HCH:(h + 1) * _HCH, :],
                dimension_numbers=(((1,), (0,)), ((), ())),
                preferred_element_type=jnp.float32,
            )
        out_ref[cb, 0, :] = acc[0] * (1.0 / z[cb * _NH, 0])
    mask_ref[...] = msk.reshape(_BC, _NH, _HCH)


def kernel(x, W, b):
    out, maskr = pl.pallas_call(
        _wta_kernel,
        grid=(_B // _BC,),
        in_specs=[
            pl.BlockSpec((_BC, _N, _DIM), lambda i: (i, 0, 0)),
            pl.BlockSpec((_DIM, _DIM), lambda i: (0, 0)),
            pl.BlockSpec((_DIM,), lambda i: (0,)),
        ],
        out_specs=[
            pl.BlockSpec((_BC, 1, _DIM), lambda i: (i, 0, 0)),
            pl.BlockSpec((_BC, _NH, _HCH), lambda i: (i, 0, 0)),
        ],
        out_shape=[
            jax.ShapeDtypeStruct((_B, 1, _DIM), jnp.float32),
            jax.ShapeDtypeStruct((_B, _NH, _HCH), jnp.float32),
        ],
    )(x, W, b)
    return out.reshape(_B, _DIM), maskr.reshape(_B, _N)
